# baseline (speedup 1.0000x reference)
import functools

import jax
import jax.numpy as jnp
from jax import lax
from jax.experimental import pallas as pl
from jax.experimental.pallas import tpu as pltpu
from jax.experimental.pallas import tpu_sc as plsc

LANES = 16
CHUNK = 64
PACK = 2


@functools.lru_cache(maxsize=None)
def _make_kernel(B, D):
    info = plsc.get_sparse_core_info()
    NC, NS = info.num_cores, info.num_subcores
    NW = NC * NS
    assert B % NW == 0 and D % LANES == 0
    b_per_w = B // NW
    mesh = plsc.VectorSubcoreMesh(core_axis_name="c", subcore_axis_name="s")

    @functools.partial(
        pl.kernel,
        mesh=mesh,
        out_type=jax.ShapeDtypeStruct((B, D), jnp.float32),
        scratch_types=[
            pltpu.VMEM((b_per_w,), jnp.int32),
            pltpu.VMEM((b_per_w,), jnp.int32),
            pltpu.VMEM((b_per_w,), jnp.int32),
            pltpu.VMEM((b_per_w,), jnp.int32),
            pltpu.VMEM((b_per_w,), jnp.int32),
            pltpu.VMEM((b_per_w,), jnp.int32),
            pltpu.VMEM((CHUNK, 1, PACK * D), jnp.float32),
            pltpu.VMEM((CHUNK, 1, PACK * D), jnp.float32),
            pltpu.VMEM((b_per_w, D), jnp.float32),
            pltpu.SemaphoreType.DMA,
            pltpu.SemaphoreType.DMA,
        ],
    )
    def gmf(uids, sids, utab3, stab3, out, uidx, sidx, urow, srow,
            uoff, soff, ublk, sblk, prod, sem_u, sem_s):
        wid = lax.axis_index("s") * NC + lax.axis_index("c")
        base = wid * b_per_w
        pltpu.sync_copy(uids.at[pl.ds(base, b_per_w)], uidx)
        pltpu.sync_copy(sids.at[pl.ds(base, b_per_w)], sidx)

        def prep(g, carry):
            sl = pl.ds(g * LANES, LANES)
            uv = uidx[sl]
            sv = sidx[sl]
            urow[sl] = uv >> 1
            srow[sl] = sv >> 1
            uoff[sl] = (uv & (PACK - 1)) * D
            soff[sl] = (sv & (PACK - 1)) * D
            return carry

        lax.fori_loop(0, b_per_w // LANES, prep, 0)

        def chunk_body(c, carry):
            lo = c * CHUNK
            for g in range(CHUNK // LANES):
                uvec = urow[pl.ds(lo + g * LANES, LANES)]
                svec = srow[pl.ds(lo + g * LANES, LANES)]
                for j in range(LANES):
                    i = g * LANES + j
                    pltpu.make_async_copy(
                        utab3.at[uvec[j]], ublk.at[i], sem_u).start()
                    pltpu.make_async_copy(
                        stab3.at[svec[j]], sblk.at[i], sem_s).start()
            for i in range(CHUNK):
                pltpu.make_async_copy(
                    utab3.at[0], ublk.at[i], sem_u).wait()
                pltpu.make_async_copy(
                    stab3.at[0], sblk.at[i], sem_s).wait()

            def body(g, carry2):
                gb = g * LANES
                uo = uoff[pl.ds(lo + gb, LANES)]
                so = soff[pl.ds(lo + gb, LANES)]
                for j in range(LANES):
                    i = gb + j
                    for k in range(D // LANES):
                        sl = pl.ds(k * LANES, LANES)
                        prod[lo + i, sl] = (
                            ublk[i, 0, pl.ds(uo[j] + k * LANES, LANES)]
                            * sblk[i, 0, pl.ds(so[j] + k * LANES, LANES)])
                return carry2

            lax.fori_loop(0, CHUNK // LANES, body, 0)
            return carry

        lax.fori_loop(0, b_per_w // CHUNK, chunk_body, 0)
        pltpu.sync_copy(prod, out.at[pl.ds(base, b_per_w)])

    return gmf


def kernel(users_ids, services_ids, user_table, service_table):
    B, = users_ids.shape
    V, D = user_table.shape
    gmf = _make_kernel(B, D)
    return gmf(
        users_ids.astype(jnp.int32),
        services_ids.astype(jnp.int32),
        user_table.reshape(V // PACK, 1, PACK * D),
        service_table.reshape(V // PACK, 1, PACK * D),
    )


# speedup vs baseline: 2.8483x; 2.8483x over previous
import functools

import jax
import jax.numpy as jnp
from jax import lax
from jax.experimental import pallas as pl
from jax.experimental.pallas import tpu as pltpu
from jax.experimental.pallas import tpu_sc as plsc

LANES = 16
CHUNK = 64


@functools.lru_cache(maxsize=None)
def _make_kernel(B, D):
    info = plsc.get_sparse_core_info()
    NC, NS = info.num_cores, info.num_subcores
    NW = NC * NS
    assert B % NW == 0 and D % LANES == 0
    b_per_w = B // NW
    assert b_per_w % (2 * CHUNK) == 0
    mesh = plsc.VectorSubcoreMesh(core_axis_name="c", subcore_axis_name="s")

    @functools.partial(
        pl.kernel,
        mesh=mesh,
        out_type=jax.ShapeDtypeStruct((B, D), jnp.float32),
        scratch_types=[
            pltpu.VMEM((b_per_w,), jnp.int32),
            pltpu.VMEM((b_per_w,), jnp.int32),
            pltpu.VMEM((CHUNK, 1, D), jnp.float32),
            pltpu.VMEM((CHUNK, 1, D), jnp.float32),
            pltpu.VMEM((CHUNK, 1, D), jnp.float32),
            pltpu.VMEM((CHUNK, 1, D), jnp.float32),
            pltpu.VMEM((b_per_w, D), jnp.float32),
            pltpu.SemaphoreType.DMA,
            pltpu.SemaphoreType.DMA,
            pltpu.SemaphoreType.DMA,
            pltpu.SemaphoreType.DMA,
        ],
    )
    def gmf(uids, sids, utab3, stab3, out, uidx, sidx,
            ublk0, sblk0, ublk1, sblk1, prod,
            sem_u0, sem_s0, sem_u1, sem_s1):
        wid = lax.axis_index("s") * NC + lax.axis_index("c")
        base = wid * b_per_w
        pltpu.sync_copy(uids.at[pl.ds(base, b_per_w)], uidx)
        pltpu.sync_copy(sids.at[pl.ds(base, b_per_w)], sidx)

        def issue(lo, ublk, sblk, sem_u, sem_s):
            for g in range(CHUNK // LANES):
                uvec = uidx[pl.ds(lo + g * LANES, LANES)]
                svec = sidx[pl.ds(lo + g * LANES, LANES)]
                for j in range(LANES):
                    i = g * LANES + j
                    pltpu.make_async_copy(
                        utab3.at[uvec[j]], ublk.at[i], sem_u).start()
                    pltpu.make_async_copy(
                        stab3.at[svec[j]], sblk.at[i], sem_s).start()

        def drain_compute(lo, ublk, sblk, sem_u, sem_s):
            pltpu.make_async_copy(
                utab3.at[pl.ds(0, CHUNK)], ublk, sem_u).wait()
            pltpu.make_async_copy(
                stab3.at[pl.ds(0, CHUNK)], sblk, sem_s).wait()

            def body(g, carry):
                gb = g * LANES
                for j in range(LANES):
                    i = gb + j
                    for k in range(D // LANES):
                        sl = pl.ds(k * LANES, LANES)
                        prod[lo + i, sl] = ublk[i, 0, sl] * sblk[i, 0, sl]
                return carry

            lax.fori_loop(0, CHUNK // LANES, body, 0)

        def pair_body(p, carry):
            lo0 = p * (2 * CHUNK)
            lo1 = lo0 + CHUNK
            issue(lo0, ublk0, sblk0, sem_u0, sem_s0)
            issue(lo1, ublk1, sblk1, sem_u1, sem_s1)
            drain_compute(lo0, ublk0, sblk0, sem_u0, sem_s0)
            drain_compute(lo1, ublk1, sblk1, sem_u1, sem_s1)
            return carry

        lax.fori_loop(0, b_per_w // (2 * CHUNK), pair_body, 0)
        pltpu.sync_copy(prod, out.at[pl.ds(base, b_per_w)])

    return gmf


def kernel(users_ids, services_ids, user_table, service_table):
    B, = users_ids.shape
    V, D = user_table.shape
    gmf = _make_kernel(B, D)
    return gmf(
        users_ids.astype(jnp.int32),
        services_ids.astype(jnp.int32),
        user_table.reshape(V, 1, D),
        service_table.reshape(V, 1, D),
    )
